# baseline (speedup 1.0000x reference)
import functools

import jax
import jax.numpy as jnp
from jax import lax
from jax.experimental import pallas as pl
from jax.experimental.pallas import tpu as pltpu
from jax.experimental.pallas import tpu_sc as plsc

N = 10000
E = 160000
H = 128
NPAD = 10240
EPAD = 163840
CH = 64
NCH = 80
SLOTS = 4
BLKS = NCH // SLOTS
KC0 = 32
KC1 = 128
KMAX = max(KC0, KC1)
RPT = NPAD // 16
TRASH = 10200
TE = 512
TN = 512
_F32 = jnp.float32


def _ln(t, g, b):
    mu = jnp.mean(t, axis=-1, keepdims=True)
    var = jnp.mean((t - mu) ** 2, axis=-1, keepdims=True)
    return (t - mu) * lax.rsqrt(var + 1e-5) * g + b


def _dot(x, w):
    return jnp.dot(x, w, preferred_element_type=_F32)


def _node_enc_body(gx, w1, b1, w2, b2, w3, b3, lg, lb, w1s, w1d,
                   hv_o, hs_o, hd_o):
    x = gx[...]
    t = x[:, 1:2] * w1[0:1, :] + x[:, 2:3] * w1[1:2, :] + b1[...]
    tp = x[:, 0:1]
    for k in range(9):
        t = t + jnp.where(tp == float(k), w1[2 + k:3 + k, :], 0.0)
    t = jax.nn.relu(t)
    t = jax.nn.relu(_dot(t, w2[...]) + b2[...])
    t = _dot(t, w3[...]) + b3[...]
    hv = _ln(t, lg[...], lb[...])
    hv_o[...] = hv
    hs_o[...] = _dot(hv, w1s[...])
    hd_o[...] = _dot(hv, w1d[...])


def _edge_enc_body(ea, w1, b1, w2, b2, w3, b3, lg, lb, he_o):
    x = ea[...]
    t = (x[:, 0:1] * w1[0:1, :] + x[:, 1:2] * w1[1:2, :]
         + x[:, 2:3] * w1[2:3, :] + b1[...])
    t = jax.nn.relu(t)
    t = jax.nn.relu(_dot(t, w2[...]) + b2[...])
    t = _dot(t, w3[...]) + b3[...]
    he_o[...] = _ln(t, lg[...], lb[...])


def _edge_mlp_body(he, g, w1e, b1, w2, b2, w3, b3, lg, lb, out):
    x = he[...]
    t = jax.nn.relu(_dot(x, w1e[...]) + g[...] + b1[...])
    t = jax.nn.relu(_dot(t, w2[...]) + b2[...])
    t = _dot(t, w3[...]) + b3[...]
    out[...] = _ln(t, lg[...], lb[...]) + x


def _node_mp_body(hv, p0, p1, wv, wa, b1, w2, b2, w3, b3, lg, lb, w1s, w1d,
                  hv_o, hs_o, hd_o):
    x = hv[...]
    agg = p0[...] + p1[...]
    t = jax.nn.relu(_dot(x, wv[...]) + _dot(agg, wa[...]) + b1[...])
    t = jax.nn.relu(_dot(t, w2[...]) + b2[...])
    t = _dot(t, w3[...]) + b3[...]
    v = _ln(t, lg[...], lb[...]) + x
    hv_o[...] = v
    hs_o[...] = _dot(v, w1s[...])
    hd_o[...] = _dot(v, w1d[...])


def _node_final_body(hv, p0, p1, fr, wv, wa, b1, w2, b2, w3, b3, lg, lb,
                     d1, db1, d2, db2, d3, db3, scale, shift, pv_o):
    x = hv[...]
    agg = p0[...] + p1[...]
    t = jax.nn.relu(_dot(x, wv[...]) + _dot(agg, wa[...]) + b1[...])
    t = jax.nn.relu(_dot(t, w2[...]) + b2[...])
    t = _dot(t, w3[...]) + b3[...]
    v = _ln(t, lg[...], lb[...]) + x
    t = jax.nn.relu(_dot(v, d1[...]) + db1[...])
    t = jax.nn.relu(_dot(t, d2[...]) + db2[...])
    o = _dot(t, d3[...]) + db3[...]
    pv_o[...] = fr[...] + o * scale[...] + shift[...]


def _row_spec(t):
    return pl.BlockSpec((t, H), lambda i: (i, 0))


def _const_spec(shape):
    nd = len(shape)
    return pl.BlockSpec(shape, lambda i: (0,) * nd)


def _small_spec(t, w):
    return pl.BlockSpec((t, w), lambda i: (i, 0))


def _node_enc_call(gx8, w1, b1, w2, b2, w3, b3, lg, lb, w1s, w1d):
    grid = (NPAD // TN,)
    hh = _const_spec((H, H))
    v = _const_spec((1, H))
    return pl.pallas_call(
        _node_enc_body,
        grid=grid,
        in_specs=[_small_spec(TN, 8), _const_spec((16, H)), v, hh, v, hh, v,
                  v, v, hh, hh],
        out_specs=[_row_spec(TN)] * 3,
        out_shape=[jax.ShapeDtypeStruct((NPAD, H), _F32)] * 3,
    )(gx8, w1, b1, w2, b2, w3, b3, lg, lb, w1s, w1d)


def _edge_enc_call(ea8, w1, b1, w2, b2, w3, b3, lg, lb):
    grid = (EPAD // TE,)
    hh = _const_spec((H, H))
    v = _const_spec((1, H))
    return pl.pallas_call(
        _edge_enc_body,
        grid=grid,
        in_specs=[_small_spec(TE, 8), _const_spec((8, H)), v, hh, v, hh, v,
                  v, v],
        out_specs=_row_spec(TE),
        out_shape=jax.ShapeDtypeStruct((EPAD, H), _F32),
    )(ea8, w1, b1, w2, b2, w3, b3, lg, lb)


def _edge_mlp_call(he, g, w1e, b1, w2, b2, w3, b3, lg, lb):
    grid = (EPAD // TE,)
    hh = _const_spec((H, H))
    v = _const_spec((1, H))
    return pl.pallas_call(
        _edge_mlp_body,
        grid=grid,
        in_specs=[_row_spec(TE)] * 2 + [hh, v, hh, v, hh, v, v, v],
        out_specs=_row_spec(TE),
        out_shape=jax.ShapeDtypeStruct((EPAD, H), _F32),
    )(he, g, w1e, b1, w2, b2, w3, b3, lg, lb)


def _node_mp_call(hv, p0, p1, wv, wa, b1, w2, b2, w3, b3, lg, lb, w1s, w1d):
    grid = (NPAD // TN,)
    hh = _const_spec((H, H))
    v = _const_spec((1, H))
    return pl.pallas_call(
        _node_mp_body,
        grid=grid,
        in_specs=[_row_spec(TN)] * 3 + [hh, hh, v, hh, v, hh, v, v, v, hh, hh],
        out_specs=[_row_spec(TN)] * 3,
        out_shape=[jax.ShapeDtypeStruct((NPAD, H), _F32)] * 3,
    )(hv, p0, p1, wv, wa, b1, w2, b2, w3, b3, lg, lb, w1s, w1d)


def _node_final_call(hv, p0, p1, fr, wv, wa, b1, w2, b2, w3, b3, lg, lb,
                     d1, db1, d2, db2, d3, db3, scale, shift):
    grid = (NPAD // TN,)
    hh = _const_spec((H, H))
    v = _const_spec((1, H))
    return pl.pallas_call(
        _node_final_body,
        grid=grid,
        in_specs=[_row_spec(TN)] * 4
        + [hh, hh, v, hh, v, hh, v, v, v, hh, v, hh, v, hh, v, v, v],
        out_specs=_row_spec(TN),
        out_shape=jax.ShapeDtypeStruct((NPAD, H), _F32),
    )(hv, p0, p1, fr, wv, wa, b1, w2, b2, w3, b3, lg, lb,
      d1, db1, d2, db2, d3, db3, scale, shift)


def _wait_write(hbm, buf, sem):
    pltpu.make_async_copy(hbm.at[pl.ds(0, CH)], buf, sem).wait()


def _add_rows(a, b):
    def body(r2, carry):
        for u in range(2):
            r = r2 * 2 + u
            for q in range(H // 16):
                sl = pl.ds(q * 16, 16)
                a[r, sl] = a[r, sl] + b[r, sl]
        return carry

    lax.fori_loop(0, CH // 2, body, 0)


def _sc_gather_body(hs_hbm, hd_hbm, sidx_hbm, didx_hbm, g_hbm,
                    sidx_v, didx_v,
                    a0, a1, a2, a3, b0, b1, b2, b3,
                    g0, g1, g2, g3, w0, w1, w2, w3):
    c = lax.axis_index("c")
    s = lax.axis_index("s")
    cbase = jnp.where(c == 0, s * KC0, 16 * KC0 + s * KC1)
    nblk = jnp.where(c == 0, KC0 // SLOTS, KC1 // SLOTS)
    pltpu.sync_copy(sidx_hbm.at[pl.ds(cbase, KMAX)], sidx_v)
    pltpu.sync_copy(didx_hbm.at[pl.ds(cbase, KMAX)], didx_v)
    abuf = (a0, a1, a2, a3)
    bbuf = (b0, b1, b2, b3)
    gsem = (g0, g1, g2, g3)
    wsem = (w0, w1, w2, w3)

    def issue(t, p):
        pltpu.async_copy(hs_hbm.at[sidx_v.at[t]], abuf[p], gsem[p])
        pltpu.async_copy(hd_hbm.at[didx_v.at[t]], bbuf[p], gsem[p])

    def wait_g(p):
        _wait_write(hs_hbm, abuf[p], gsem[p])
        _wait_write(hs_hbm, bbuf[p], gsem[p])

    def flush(t, p):
        wait_g(p)
        _add_rows(abuf[p], bbuf[p])
        pltpu.async_copy(abuf[p], g_hbm.at[pl.ds((cbase + t) * CH, CH)],
                         wsem[p])

    for p in range(SLOTS):
        issue(p, p)

    def body(jj, carry):
        for p in range(SLOTS):
            flush(jj * SLOTS + p, p)
        for p in range(SLOTS):
            _wait_write(g_hbm, abuf[p], wsem[p])
            issue(jj * SLOTS + p + SLOTS, p)
        return carry

    lax.fori_loop(0, nblk - 1, body, 0)
    for p in range(SLOTS):
        flush((nblk - 1) * SLOTS + p, p)
    for p in range(SLOTS):
        _wait_write(g_hbm, abuf[p], wsem[p])


def _sc_scatter_body(enew_hbm, didx_hbm, zeros_hbm, out_hbm,
                     didx_v, r0, r1, r2, r3, acc,
                     rs0, rs1, rs2, rs3, as0, as1, as2, as3):
    c = lax.axis_index("c")
    s = lax.axis_index("s")
    wid = s * 2 + c
    pltpu.sync_copy(didx_hbm.at[pl.ds(wid * NCH, NCH)], didx_v)
    base = wid * NCH * CH
    rbuf = (r0, r1, r2, r3)
    rsem = (rs0, rs1, rs2, rs3)
    asem = (as0, as1, as2, as3)

    def issue_read(t, p):
        pltpu.async_copy(enew_hbm.at[pl.ds(base + t * CH, CH)], rbuf[p],
                         rsem[p])

    for p in range(SLOTS):
        issue_read(p, p)
    pltpu.sync_copy(zeros_hbm, acc.at[pl.ds(s * RPT, RPT)])
    plsc.subcore_barrier()

    def body(jj, carry):
        for p in range(SLOTS):
            t = jj * SLOTS + p
            _wait_write(enew_hbm, rbuf[p], rsem[p])
            pltpu.async_copy(rbuf[p], acc.at[didx_v.at[t]], asem[p],
                             add=True)
        for p in range(SLOTS):
            _wait_write(enew_hbm, rbuf[p], asem[p])
            issue_read(jj * SLOTS + p + SLOTS, p)
        return carry

    lax.fori_loop(0, BLKS - 1, body, 0)
    for p in range(SLOTS):
        t = (BLKS - 1) * SLOTS + p
        _wait_write(enew_hbm, rbuf[p], rsem[p])
        pltpu.async_copy(rbuf[p], acc.at[didx_v.at[t]], asem[p], add=True)
    for p in range(SLOTS):
        _wait_write(enew_hbm, rbuf[p], asem[p])
    plsc.subcore_barrier()
    pltpu.sync_copy(acc.at[pl.ds(s * RPT, RPT)],
                    out_hbm.at[c, pl.ds(s * RPT, RPT)])


@functools.lru_cache(maxsize=None)
def _sc_gather_call():
    return functools.partial(
        pl.kernel,
        mesh=plsc.VectorSubcoreMesh(core_axis_name="c", subcore_axis_name="s"),
        out_type=jax.ShapeDtypeStruct((EPAD, H), _F32),
        scratch_types=[pltpu.VMEM((KMAX, CH), jnp.int32),
                       pltpu.VMEM((KMAX, CH), jnp.int32)]
        + [pltpu.VMEM((CH, H), _F32)] * (2 * SLOTS)
        + [pltpu.SemaphoreType.DMA] * (2 * SLOTS),
    )(_sc_gather_body)


@functools.lru_cache(maxsize=None)
def _sc_scatter_call():
    return functools.partial(
        pl.kernel,
        mesh=plsc.VectorSubcoreMesh(core_axis_name="c", subcore_axis_name="s"),
        out_type=jax.ShapeDtypeStruct((2, NPAD, H), _F32),
        scratch_types=[pltpu.VMEM((NCH, CH), jnp.int32)]
        + [pltpu.VMEM((CH, H), _F32)] * SLOTS
        + [pltpu.VMEM_SHARED((NPAD, H), _F32)]
        + [pltpu.SemaphoreType.DMA] * (2 * SLOTS),
    )(_sc_scatter_body)


def kernel(graph_x, edge_index, edge_attr, velocity_sequence_noise,
           enc_node, enc_edge, mp_edge, mp_node, dec, norm_stats):
    del velocity_sequence_noise
    node_mean, node_std, edge_mean, edge_std, out_mean, out_std = norm_stats
    mp = len(mp_edge)

    src = edge_index[0].astype(jnp.int32)
    dst = edge_index[1].astype(jnp.int32)
    pad_e = EPAD - E
    sidx = jnp.concatenate(
        [src, jnp.full((pad_e,), TRASH, jnp.int32)]).reshape(EPAD // CH, CH)
    didx = jnp.concatenate(
        [dst, jnp.full((pad_e,), TRASH, jnp.int32)]).reshape(EPAD // CH, CH)
    ipad = jnp.full((KMAX, CH), TRASH, jnp.int32)
    sidx = jnp.concatenate([sidx, ipad])
    didx = jnp.concatenate([didx, ipad])

    gx8 = jnp.zeros((NPAD, 8), _F32).at[:N, :3].set(graph_x)
    ea8 = jnp.zeros((EPAD, 8), _F32).at[:E, :3].set(edge_attr)
    frames_pad = jnp.zeros((NPAD, H), _F32).at[:N, :2].set(graph_x[:, 1:3])
    zrows = jnp.zeros((RPT, H), _F32)

    def r(v):
        return v.reshape(1, H)

    nw1, nb1, nw2, nb2, nw3, nb3, nlg, nlb = enc_node
    ninv = 1.0 / node_std
    nw1p = jnp.zeros((16, H), _F32).at[:11].set(nw1 * ninv[:, None])
    nb1p = nb1 - (node_mean * ninv) @ nw1

    ew1, eb1, ew2, eb2, ew3, eb3, elg, elb = enc_edge
    einv = 1.0 / edge_std
    ew1p = jnp.zeros((8, H), _F32).at[:3].set(ew1 * einv[:, None])
    eb1p = eb1 - (edge_mean * einv) @ ew1

    d1, db1, d2, db2, d3, db3 = dec
    d3p = jnp.zeros((H, H), _F32).at[:, :2].set(d3)
    db3p = jnp.zeros((H,), _F32).at[:2].set(db3)
    scale = jnp.zeros((1, H), _F32).at[0, :2].set(out_std)
    shift = jnp.zeros((1, H), _F32).at[0, :2].set(out_mean)

    w1s0 = mp_edge[0][0][H:2 * H]
    w1d0 = mp_edge[0][0][2 * H:3 * H]
    hv, hs, hd = _node_enc_call(gx8, nw1p, r(nb1p), nw2, r(nb2), nw3, r(nb3),
                                r(nlg), r(nlb), w1s0, w1d0)
    he = _edge_enc_call(ea8, ew1p, r(eb1p), ew2, r(eb2), ew3, r(eb3),
                        r(elg), r(elb))

    pv = None
    for i in range(mp):
        we = mp_edge[i]
        wn = mp_node[i]
        g = _sc_gather_call()(hs, hd, sidx, didx)
        he = _edge_mlp_call(he, g, we[0][:H], r(we[1]), we[2], r(we[3]),
                            we[4], r(we[5]), r(we[6]), r(we[7]))
        parts = _sc_scatter_call()(he, didx, zrows)
        wv = wn[0][:H]
        wa = wn[0][H:]
        if i < mp - 1:
            w1s = mp_edge[i + 1][0][H:2 * H]
            w1d = mp_edge[i + 1][0][2 * H:3 * H]
            hv, hs, hd = _node_mp_call(hv, parts[0], parts[1], wv, wa,
                                       r(wn[1]), wn[2], r(wn[3]), wn[4],
                                       r(wn[5]), r(wn[6]), r(wn[7]),
                                       w1s, w1d)
        else:
            pv = _node_final_call(hv, parts[0], parts[1], frames_pad, wv, wa,
                                  r(wn[1]), wn[2], r(wn[3]), wn[4], r(wn[5]),
                                  r(wn[6]), r(wn[7]), d1, r(db1), d2, r(db2),
                                  d3p, r(db3p), scale, shift)

    return pv[:N, :2]


# speedup vs baseline: 1.1077x; 1.1077x over previous
import functools

import jax
import jax.numpy as jnp
from jax import lax
from jax.experimental import pallas as pl
from jax.experimental.pallas import tpu as pltpu
from jax.experimental.pallas import tpu_sc as plsc

N = 10000
E = 160000
H = 128
NPAD = 10240
EPAD = 163840
CH = 64
NCH = 80
SLOTS = 4
BLKS = NCH // SLOTS
KC0 = 128
KC1 = 32
KMAX = max(KC0, KC1)
RPT = NPAD // 16
TRASH = 10200
TE = 512
TN = 512
_F32 = jnp.float32


def _ln(t, g, b):
    mu = jnp.mean(t, axis=-1, keepdims=True)
    var = jnp.mean((t - mu) ** 2, axis=-1, keepdims=True)
    return (t - mu) * lax.rsqrt(var + 1e-5) * g + b


def _dot(x, w):
    return jnp.dot(x, w, preferred_element_type=_F32)


def _node_enc_body(gx, w1, b1, w2, b2, w3, b3, lg, lb, w1s, w1d,
                   hv_o, hs_o, hd_o):
    x = gx[...]
    t = x[:, 1:2] * w1[0:1, :] + x[:, 2:3] * w1[1:2, :] + b1[...]
    tp = x[:, 0:1]
    for k in range(9):
        t = t + jnp.where(tp == float(k), w1[2 + k:3 + k, :], 0.0)
    t = jax.nn.relu(t)
    t = jax.nn.relu(_dot(t, w2[...]) + b2[...])
    t = _dot(t, w3[...]) + b3[...]
    hv = _ln(t, lg[...], lb[...])
    hv_o[...] = hv
    hs_o[...] = _dot(hv, w1s[...])
    hd_o[...] = _dot(hv, w1d[...])


def _edge_enc_body(ea, w1, b1, w2, b2, w3, b3, lg, lb, he_o):
    x = ea[...]
    t = (x[:, 0:1] * w1[0:1, :] + x[:, 1:2] * w1[1:2, :]
         + x[:, 2:3] * w1[2:3, :] + b1[...])
    t = jax.nn.relu(t)
    t = jax.nn.relu(_dot(t, w2[...]) + b2[...])
    t = _dot(t, w3[...]) + b3[...]
    he_o[...] = _ln(t, lg[...], lb[...])


def _edge_mlp_body(he, g, w1e, b1, w2, b2, w3, b3, lg, lb, out):
    x = he[...]
    t = jax.nn.relu(_dot(x, w1e[...]) + g[...] + b1[...])
    t = jax.nn.relu(_dot(t, w2[...]) + b2[...])
    t = _dot(t, w3[...]) + b3[...]
    out[...] = _ln(t, lg[...], lb[...]) + x


def _node_mp_body(hv, p0, p1, wv, wa, b1, w2, b2, w3, b3, lg, lb, w1s, w1d,
                  hv_o, hs_o, hd_o):
    x = hv[...]
    agg = p0[...] + p1[...]
    t = jax.nn.relu(_dot(x, wv[...]) + _dot(agg, wa[...]) + b1[...])
    t = jax.nn.relu(_dot(t, w2[...]) + b2[...])
    t = _dot(t, w3[...]) + b3[...]
    v = _ln(t, lg[...], lb[...]) + x
    hv_o[...] = v
    hs_o[...] = _dot(v, w1s[...])
    hd_o[...] = _dot(v, w1d[...])


def _node_final_body(hv, p0, p1, fr, wv, wa, b1, w2, b2, w3, b3, lg, lb,
                     d1, db1, d2, db2, d3, db3, scale, shift, pv_o):
    x = hv[...]
    agg = p0[...] + p1[...]
    t = jax.nn.relu(_dot(x, wv[...]) + _dot(agg, wa[...]) + b1[...])
    t = jax.nn.relu(_dot(t, w2[...]) + b2[...])
    t = _dot(t, w3[...]) + b3[...]
    v = _ln(t, lg[...], lb[...]) + x
    t = jax.nn.relu(_dot(v, d1[...]) + db1[...])
    t = jax.nn.relu(_dot(t, d2[...]) + db2[...])
    o = _dot(t, d3[...]) + db3[...]
    pv_o[...] = fr[...] + o * scale[...] + shift[...]


def _row_spec(t):
    return pl.BlockSpec((t, H), lambda i: (i, 0))


def _const_spec(shape):
    nd = len(shape)
    return pl.BlockSpec(shape, lambda i: (0,) * nd)


def _small_spec(t, w):
    return pl.BlockSpec((t, w), lambda i: (i, 0))


def _node_enc_call(gx8, w1, b1, w2, b2, w3, b3, lg, lb, w1s, w1d):
    grid = (NPAD // TN,)
    hh = _const_spec((H, H))
    v = _const_spec((1, H))
    return pl.pallas_call(
        _node_enc_body,
        grid=grid,
        in_specs=[_small_spec(TN, 8), _const_spec((16, H)), v, hh, v, hh, v,
                  v, v, hh, hh],
        out_specs=[_row_spec(TN)] * 3,
        out_shape=[jax.ShapeDtypeStruct((NPAD, H), _F32)] * 3,
    )(gx8, w1, b1, w2, b2, w3, b3, lg, lb, w1s, w1d)


def _edge_enc_call(ea8, w1, b1, w2, b2, w3, b3, lg, lb):
    grid = (EPAD // TE,)
    hh = _const_spec((H, H))
    v = _const_spec((1, H))
    return pl.pallas_call(
        _edge_enc_body,
        grid=grid,
        in_specs=[_small_spec(TE, 8), _const_spec((8, H)), v, hh, v, hh, v,
                  v, v],
        out_specs=_row_spec(TE),
        out_shape=jax.ShapeDtypeStruct((EPAD, H), _F32),
    )(ea8, w1, b1, w2, b2, w3, b3, lg, lb)


def _edge_mlp_call(he, g, w1e, b1, w2, b2, w3, b3, lg, lb):
    grid = (EPAD // TE,)
    hh = _const_spec((H, H))
    v = _const_spec((1, H))
    return pl.pallas_call(
        _edge_mlp_body,
        grid=grid,
        in_specs=[_row_spec(TE)] * 2 + [hh, v, hh, v, hh, v, v, v],
        out_specs=_row_spec(TE),
        out_shape=jax.ShapeDtypeStruct((EPAD, H), _F32),
    )(he, g, w1e, b1, w2, b2, w3, b3, lg, lb)


def _node_mp_call(hv, p0, p1, wv, wa, b1, w2, b2, w3, b3, lg, lb, w1s, w1d):
    grid = (NPAD // TN,)
    hh = _const_spec((H, H))
    v = _const_spec((1, H))
    return pl.pallas_call(
        _node_mp_body,
        grid=grid,
        in_specs=[_row_spec(TN)] * 3 + [hh, hh, v, hh, v, hh, v, v, v, hh, hh],
        out_specs=[_row_spec(TN)] * 3,
        out_shape=[jax.ShapeDtypeStruct((NPAD, H), _F32)] * 3,
    )(hv, p0, p1, wv, wa, b1, w2, b2, w3, b3, lg, lb, w1s, w1d)


def _node_final_call(hv, p0, p1, fr, wv, wa, b1, w2, b2, w3, b3, lg, lb,
                     d1, db1, d2, db2, d3, db3, scale, shift):
    grid = (NPAD // TN,)
    hh = _const_spec((H, H))
    v = _const_spec((1, H))
    return pl.pallas_call(
        _node_final_body,
        grid=grid,
        in_specs=[_row_spec(TN)] * 4
        + [hh, hh, v, hh, v, hh, v, v, v, hh, v, hh, v, hh, v, v, v],
        out_specs=_row_spec(TN),
        out_shape=jax.ShapeDtypeStruct((NPAD, H), _F32),
    )(hv, p0, p1, fr, wv, wa, b1, w2, b2, w3, b3, lg, lb,
      d1, db1, d2, db2, d3, db3, scale, shift)


def _wait_write(hbm, buf, sem):
    pltpu.make_async_copy(hbm.at[pl.ds(0, CH)], buf, sem).wait()


def _add_rows(a, b):
    def body(r2, carry):
        for u in range(2):
            r = r2 * 2 + u
            for q in range(H // 16):
                sl = pl.ds(q * 16, 16)
                a[r, sl] = a[r, sl] + b[r, sl]
        return carry

    lax.fori_loop(0, CH // 2, body, 0)


def _sc_gather_body(hs_hbm, hd_hbm, sidx_hbm, didx_hbm, g_hbm,
                    sidx_v, didx_v,
                    a0, a1, a2, a3, b0, b1, b2, b3,
                    g0, g1, g2, g3, w0, w1, w2, w3):
    c = lax.axis_index("c")
    s = lax.axis_index("s")
    cbase = jnp.where(c == 0, s * KC0, 16 * KC0 + s * KC1)
    nblk = jnp.where(c == 0, KC0 // SLOTS, KC1 // SLOTS)
    pltpu.sync_copy(sidx_hbm.at[pl.ds(cbase, KMAX)], sidx_v)
    pltpu.sync_copy(didx_hbm.at[pl.ds(cbase, KMAX)], didx_v)
    abuf = (a0, a1, a2, a3)
    bbuf = (b0, b1, b2, b3)
    gsem = (g0, g1, g2, g3)
    wsem = (w0, w1, w2, w3)

    def issue(t, p):
        pltpu.async_copy(hs_hbm.at[sidx_v.at[t]], abuf[p], gsem[p])
        pltpu.async_copy(hd_hbm.at[didx_v.at[t]], bbuf[p], gsem[p])

    def wait_g(p):
        _wait_write(hs_hbm, abuf[p], gsem[p])
        _wait_write(hs_hbm, bbuf[p], gsem[p])

    def flush(t, p):
        wait_g(p)
        _add_rows(abuf[p], bbuf[p])
        pltpu.async_copy(abuf[p], g_hbm.at[pl.ds((cbase + t) * CH, CH)],
                         wsem[p])

    for p in range(SLOTS):
        issue(p, p)

    def body(jj, carry):
        for p in range(SLOTS):
            flush(jj * SLOTS + p, p)
        for p in range(SLOTS):
            _wait_write(g_hbm, abuf[p], wsem[p])
            issue(jj * SLOTS + p + SLOTS, p)
        return carry

    lax.fori_loop(0, nblk - 1, body, 0)
    for p in range(SLOTS):
        flush((nblk - 1) * SLOTS + p, p)
    for p in range(SLOTS):
        _wait_write(g_hbm, abuf[p], wsem[p])


def _sc_scatter_body(enew_hbm, didx_hbm, zeros_hbm, out_hbm,
                     didx_v, r0, r1, r2, r3, acc,
                     rs0, rs1, rs2, rs3, as0, as1, as2, as3):
    c = lax.axis_index("c")
    s = lax.axis_index("s")
    wid = s * 2 + c
    pltpu.sync_copy(didx_hbm.at[pl.ds(wid * NCH, NCH)], didx_v)
    base = wid * NCH * CH
    rbuf = (r0, r1, r2, r3)
    rsem = (rs0, rs1, rs2, rs3)
    asem = (as0, as1, as2, as3)

    def issue_read(t, p):
        pltpu.async_copy(enew_hbm.at[pl.ds(base + t * CH, CH)], rbuf[p],
                         rsem[p])

    for p in range(SLOTS):
        issue_read(p, p)
    pltpu.sync_copy(zeros_hbm, acc.at[pl.ds(s * RPT, RPT)])
    plsc.subcore_barrier()

    def body(jj, carry):
        for p in range(SLOTS):
            t = jj * SLOTS + p
            _wait_write(enew_hbm, rbuf[p], rsem[p])
            pltpu.async_copy(rbuf[p], acc.at[didx_v.at[t]], asem[p],
                             add=True)
        for p in range(SLOTS):
            _wait_write(enew_hbm, rbuf[p], asem[p])
            issue_read(jj * SLOTS + p + SLOTS, p)
        return carry

    lax.fori_loop(0, BLKS - 1, body, 0)
    for p in range(SLOTS):
        t = (BLKS - 1) * SLOTS + p
        _wait_write(enew_hbm, rbuf[p], rsem[p])
        pltpu.async_copy(rbuf[p], acc.at[didx_v.at[t]], asem[p], add=True)
    for p in range(SLOTS):
        _wait_write(enew_hbm, rbuf[p], asem[p])
    plsc.subcore_barrier()
    pltpu.sync_copy(acc.at[pl.ds(s * RPT, RPT)],
                    out_hbm.at[c, pl.ds(s * RPT, RPT)])


@functools.lru_cache(maxsize=None)
def _sc_gather_call():
    return functools.partial(
        pl.kernel,
        mesh=plsc.VectorSubcoreMesh(core_axis_name="c", subcore_axis_name="s"),
        out_type=jax.ShapeDtypeStruct((EPAD, H), _F32),
        scratch_types=[pltpu.VMEM((KMAX, CH), jnp.int32),
                       pltpu.VMEM((KMAX, CH), jnp.int32)]
        + [pltpu.VMEM((CH, H), _F32)] * (2 * SLOTS)
        + [pltpu.SemaphoreType.DMA] * (2 * SLOTS),
    )(_sc_gather_body)


@functools.lru_cache(maxsize=None)
def _sc_scatter_call():
    return functools.partial(
        pl.kernel,
        mesh=plsc.VectorSubcoreMesh(core_axis_name="c", subcore_axis_name="s"),
        out_type=jax.ShapeDtypeStruct((2, NPAD, H), _F32),
        scratch_types=[pltpu.VMEM((NCH, CH), jnp.int32)]
        + [pltpu.VMEM((CH, H), _F32)] * SLOTS
        + [pltpu.VMEM_SHARED((NPAD, H), _F32)]
        + [pltpu.SemaphoreType.DMA] * (2 * SLOTS),
    )(_sc_scatter_body)


def kernel(graph_x, edge_index, edge_attr, velocity_sequence_noise,
           enc_node, enc_edge, mp_edge, mp_node, dec, norm_stats):
    del velocity_sequence_noise
    node_mean, node_std, edge_mean, edge_std, out_mean, out_std = norm_stats
    mp = len(mp_edge)

    src = edge_index[0].astype(jnp.int32)
    dst = edge_index[1].astype(jnp.int32)
    pad_e = EPAD - E
    sidx = jnp.concatenate(
        [src, jnp.full((pad_e,), TRASH, jnp.int32)]).reshape(EPAD // CH, CH)
    didx = jnp.concatenate(
        [dst, jnp.full((pad_e,), TRASH, jnp.int32)]).reshape(EPAD // CH, CH)
    ipad = jnp.full((KMAX, CH), TRASH, jnp.int32)
    sidx = jnp.concatenate([sidx, ipad])
    didx = jnp.concatenate([didx, ipad])

    gx8 = jnp.zeros((NPAD, 8), _F32).at[:N, :3].set(graph_x)
    ea8 = jnp.zeros((EPAD, 8), _F32).at[:E, :3].set(edge_attr)
    frames_pad = jnp.zeros((NPAD, H), _F32).at[:N, :2].set(graph_x[:, 1:3])
    zrows = jnp.zeros((RPT, H), _F32)

    def r(v):
        return v.reshape(1, H)

    nw1, nb1, nw2, nb2, nw3, nb3, nlg, nlb = enc_node
    ninv = 1.0 / node_std
    nw1p = jnp.zeros((16, H), _F32).at[:11].set(nw1 * ninv[:, None])
    nb1p = nb1 - (node_mean * ninv) @ nw1

    ew1, eb1, ew2, eb2, ew3, eb3, elg, elb = enc_edge
    einv = 1.0 / edge_std
    ew1p = jnp.zeros((8, H), _F32).at[:3].set(ew1 * einv[:, None])
    eb1p = eb1 - (edge_mean * einv) @ ew1

    d1, db1, d2, db2, d3, db3 = dec
    d3p = jnp.zeros((H, H), _F32).at[:, :2].set(d3)
    db3p = jnp.zeros((H,), _F32).at[:2].set(db3)
    scale = jnp.zeros((1, H), _F32).at[0, :2].set(out_std)
    shift = jnp.zeros((1, H), _F32).at[0, :2].set(out_mean)

    w1s0 = mp_edge[0][0][H:2 * H]
    w1d0 = mp_edge[0][0][2 * H:3 * H]
    hv, hs, hd = _node_enc_call(gx8, nw1p, r(nb1p), nw2, r(nb2), nw3, r(nb3),
                                r(nlg), r(nlb), w1s0, w1d0)
    he = _edge_enc_call(ea8, ew1p, r(eb1p), ew2, r(eb2), ew3, r(eb3),
                        r(elg), r(elb))

    pv = None
    for i in range(mp):
        we = mp_edge[i]
        wn = mp_node[i]
        g = _sc_gather_call()(hs, hd, sidx, didx)
        he = _edge_mlp_call(he, g, we[0][:H], r(we[1]), we[2], r(we[3]),
                            we[4], r(we[5]), r(we[6]), r(we[7]))
        parts = _sc_scatter_call()(he, didx, zrows)
        wv = wn[0][:H]
        wa = wn[0][H:]
        if i < mp - 1:
            w1s = mp_edge[i + 1][0][H:2 * H]
            w1d = mp_edge[i + 1][0][2 * H:3 * H]
            hv, hs, hd = _node_mp_call(hv, parts[0], parts[1], wv, wa,
                                       r(wn[1]), wn[2], r(wn[3]), wn[4],
                                       r(wn[5]), r(wn[6]), r(wn[7]),
                                       w1s, w1d)
        else:
            pv = _node_final_call(hv, parts[0], parts[1], frames_pad, wv, wa,
                                  r(wn[1]), wn[2], r(wn[3]), wn[4], r(wn[5]),
                                  r(wn[6]), r(wn[7]), d1, r(db1), d2, r(db2),
                                  d3p, r(db3p), scale, shift)

    return pv[:N, :2]


# speedup vs baseline: 1.2406x; 1.1200x over previous
import functools

import jax
import jax.numpy as jnp
from jax import lax
from jax.experimental import pallas as pl
from jax.experimental.pallas import tpu as pltpu
from jax.experimental.pallas import tpu_sc as plsc

N = 10000
E = 160000
H = 128
NPAD = 10240
EPAD = 163840
CH = 64
NCH = 80
SLOTS = 4
BLKS = NCH // SLOTS
KC0 = 112
KC1 = 48
KMAX = max(KC0, KC1)
RPT = NPAD // 16
TRASH = 10200
TE = 1024
TN = 512
_F32 = jnp.float32


def _ln(t, g, b):
    mu = jnp.mean(t, axis=-1, keepdims=True)
    var = jnp.mean((t - mu) ** 2, axis=-1, keepdims=True)
    return (t - mu) * lax.rsqrt(var + 1e-5) * g + b


def _dot(x, w):
    return jnp.dot(x.astype(jnp.bfloat16), w.astype(jnp.bfloat16),
                   preferred_element_type=_F32)


def _node_enc_body(gx, w1, b1, w2, b2, w3, b3, lg, lb, w1s, w1d,
                   hv_o, hs_o, hd_o):
    x = gx[...]
    t = x[:, 1:2] * w1[0:1, :] + x[:, 2:3] * w1[1:2, :] + b1[...]
    tp = x[:, 0:1]
    for k in range(9):
        t = t + jnp.where(tp == float(k), w1[2 + k:3 + k, :], 0.0)
    t = jax.nn.relu(t)
    t = jax.nn.relu(_dot(t, w2[...]) + b2[...])
    t = _dot(t, w3[...]) + b3[...]
    hv = _ln(t, lg[...], lb[...])
    hv_o[...] = hv
    hs_o[...] = _dot(hv, w1s[...])
    hd_o[...] = _dot(hv, w1d[...])


def _edge_enc_body(ea, w1, b1, w2, b2, w3, b3, lg, lb, he_o):
    x = ea[...]
    t = (x[:, 0:1] * w1[0:1, :] + x[:, 1:2] * w1[1:2, :]
         + x[:, 2:3] * w1[2:3, :] + b1[...])
    t = jax.nn.relu(t)
    t = jax.nn.relu(_dot(t, w2[...]) + b2[...])
    t = _dot(t, w3[...]) + b3[...]
    he_o[...] = _ln(t, lg[...], lb[...])


def _edge_mlp_body(he, g, w1e, b1, w2, b2, w3, b3, lg, lb, out):
    x = he[...]
    t = jax.nn.relu(_dot(x, w1e[...]) + g[...] + b1[...])
    t = jax.nn.relu(_dot(t, w2[...]) + b2[...])
    t = _dot(t, w3[...]) + b3[...]
    out[...] = _ln(t, lg[...], lb[...]) + x


def _node_mp_body(hv, p0, p1, wv, wa, b1, w2, b2, w3, b3, lg, lb, w1s, w1d,
                  hv_o, hs_o, hd_o):
    x = hv[...]
    agg = p0[...] + p1[...]
    t = jax.nn.relu(_dot(x, wv[...]) + _dot(agg, wa[...]) + b1[...])
    t = jax.nn.relu(_dot(t, w2[...]) + b2[...])
    t = _dot(t, w3[...]) + b3[...]
    v = _ln(t, lg[...], lb[...]) + x
    hv_o[...] = v
    hs_o[...] = _dot(v, w1s[...])
    hd_o[...] = _dot(v, w1d[...])


def _node_final_body(hv, p0, p1, fr, wv, wa, b1, w2, b2, w3, b3, lg, lb,
                     d1, db1, d2, db2, d3, db3, scale, shift, pv_o):
    x = hv[...]
    agg = p0[...] + p1[...]
    t = jax.nn.relu(_dot(x, wv[...]) + _dot(agg, wa[...]) + b1[...])
    t = jax.nn.relu(_dot(t, w2[...]) + b2[...])
    t = _dot(t, w3[...]) + b3[...]
    v = _ln(t, lg[...], lb[...]) + x
    t = jax.nn.relu(_dot(v, d1[...]) + db1[...])
    t = jax.nn.relu(_dot(t, d2[...]) + db2[...])
    o = _dot(t, d3[...]) + db3[...]
    pv_o[...] = fr[...] + o * scale[...] + shift[...]


def _row_spec(t):
    return pl.BlockSpec((t, H), lambda i: (i, 0))


def _const_spec(shape):
    nd = len(shape)
    return pl.BlockSpec(shape, lambda i: (0,) * nd)


def _small_spec(t, w):
    return pl.BlockSpec((t, w), lambda i: (i, 0))


def _node_enc_call(gx8, w1, b1, w2, b2, w3, b3, lg, lb, w1s, w1d):
    grid = (NPAD // TN,)
    hh = _const_spec((H, H))
    v = _const_spec((1, H))
    return pl.pallas_call(
        _node_enc_body,
        grid=grid,
        in_specs=[_small_spec(TN, 8), _const_spec((16, H)), v, hh, v, hh, v,
                  v, v, hh, hh],
        out_specs=[_row_spec(TN)] * 3,
        out_shape=[jax.ShapeDtypeStruct((NPAD, H), _F32)] * 3,
    )(gx8, w1, b1, w2, b2, w3, b3, lg, lb, w1s, w1d)


def _edge_enc_call(ea8, w1, b1, w2, b2, w3, b3, lg, lb):
    grid = (EPAD // TE,)
    hh = _const_spec((H, H))
    v = _const_spec((1, H))
    return pl.pallas_call(
        _edge_enc_body,
        grid=grid,
        in_specs=[_small_spec(TE, 8), _const_spec((8, H)), v, hh, v, hh, v,
                  v, v],
        out_specs=_row_spec(TE),
        out_shape=jax.ShapeDtypeStruct((EPAD, H), _F32),
    )(ea8, w1, b1, w2, b2, w3, b3, lg, lb)


def _edge_mlp_call(he, g, w1e, b1, w2, b2, w3, b3, lg, lb):
    grid = (EPAD // TE,)
    hh = _const_spec((H, H))
    v = _const_spec((1, H))
    return pl.pallas_call(
        _edge_mlp_body,
        grid=grid,
        in_specs=[_row_spec(TE)] * 2 + [hh, v, hh, v, hh, v, v, v],
        out_specs=_row_spec(TE),
        out_shape=jax.ShapeDtypeStruct((EPAD, H), _F32),
    )(he, g, w1e, b1, w2, b2, w3, b3, lg, lb)


def _node_mp_call(hv, p0, p1, wv, wa, b1, w2, b2, w3, b3, lg, lb, w1s, w1d):
    grid = (NPAD // TN,)
    hh = _const_spec((H, H))
    v = _const_spec((1, H))
    return pl.pallas_call(
        _node_mp_body,
        grid=grid,
        in_specs=[_row_spec(TN)] * 3 + [hh, hh, v, hh, v, hh, v, v, v, hh, hh],
        out_specs=[_row_spec(TN)] * 3,
        out_shape=[jax.ShapeDtypeStruct((NPAD, H), _F32)] * 3,
    )(hv, p0, p1, wv, wa, b1, w2, b2, w3, b3, lg, lb, w1s, w1d)


def _node_final_call(hv, p0, p1, fr, wv, wa, b1, w2, b2, w3, b3, lg, lb,
                     d1, db1, d2, db2, d3, db3, scale, shift):
    grid = (NPAD // TN,)
    hh = _const_spec((H, H))
    v = _const_spec((1, H))
    return pl.pallas_call(
        _node_final_body,
        grid=grid,
        in_specs=[_row_spec(TN)] * 4
        + [hh, hh, v, hh, v, hh, v, v, v, hh, v, hh, v, hh, v, v, v],
        out_specs=_row_spec(TN),
        out_shape=jax.ShapeDtypeStruct((NPAD, H), _F32),
    )(hv, p0, p1, fr, wv, wa, b1, w2, b2, w3, b3, lg, lb,
      d1, db1, d2, db2, d3, db3, scale, shift)


def _wait_write(hbm, buf, sem):
    pltpu.make_async_copy(hbm.at[pl.ds(0, CH)], buf, sem).wait()


def _add_rows(a, b):
    def body(r2, carry):
        for u in range(2):
            r = r2 * 2 + u
            for q in range(H // 16):
                sl = pl.ds(q * 16, 16)
                a[r, sl] = a[r, sl] + b[r, sl]
        return carry

    lax.fori_loop(0, CH // 2, body, 0)


def _sc_gather_body(hs_hbm, hd_hbm, sidx_hbm, didx_hbm, g_hbm,
                    sidx_v, didx_v,
                    a0, a1, a2, a3, b0, b1, b2, b3,
                    g0, g1, g2, g3, w0, w1, w2, w3):
    c = lax.axis_index("c")
    s = lax.axis_index("s")
    cbase = jnp.where(c == 0, s * KC0, 16 * KC0 + s * KC1)
    nblk = jnp.where(c == 0, KC0 // SLOTS, KC1 // SLOTS)
    pltpu.sync_copy(sidx_hbm.at[pl.ds(cbase, KMAX)], sidx_v)
    pltpu.sync_copy(didx_hbm.at[pl.ds(cbase, KMAX)], didx_v)
    abuf = (a0, a1, a2, a3)
    bbuf = (b0, b1, b2, b3)
    gsem = (g0, g1, g2, g3)
    wsem = (w0, w1, w2, w3)

    def issue(t, p):
        pltpu.async_copy(hs_hbm.at[sidx_v.at[t]], abuf[p], gsem[p])
        pltpu.async_copy(hd_hbm.at[didx_v.at[t]], bbuf[p], gsem[p])

    def wait_g(p):
        _wait_write(hs_hbm, abuf[p], gsem[p])
        _wait_write(hs_hbm, bbuf[p], gsem[p])

    def flush(t, p):
        wait_g(p)
        _add_rows(abuf[p], bbuf[p])
        pltpu.async_copy(abuf[p], g_hbm.at[pl.ds((cbase + t) * CH, CH)],
                         wsem[p])

    for p in range(SLOTS):
        issue(p, p)

    def body(jj, carry):
        for p in range(SLOTS):
            flush(jj * SLOTS + p, p)
        for p in range(SLOTS):
            _wait_write(g_hbm, abuf[p], wsem[p])
            issue(jj * SLOTS + p + SLOTS, p)
        return carry

    lax.fori_loop(0, nblk - 1, body, 0)
    for p in range(SLOTS):
        flush((nblk - 1) * SLOTS + p, p)
    for p in range(SLOTS):
        _wait_write(g_hbm, abuf[p], wsem[p])


def _sc_scatter_body(enew_hbm, didx_hbm, zeros_hbm, out_hbm,
                     didx_v, r0, r1, r2, r3, acc,
                     rs0, rs1, rs2, rs3, as0, as1, as2, as3):
    c = lax.axis_index("c")
    s = lax.axis_index("s")
    wid = s * 2 + c
    pltpu.sync_copy(didx_hbm.at[pl.ds(wid * NCH, NCH)], didx_v)
    base = wid * NCH * CH
    rbuf = (r0, r1, r2, r3)
    rsem = (rs0, rs1, rs2, rs3)
    asem = (as0, as1, as2, as3)

    def issue_read(t, p):
        pltpu.async_copy(enew_hbm.at[pl.ds(base + t * CH, CH)], rbuf[p],
                         rsem[p])

    for p in range(SLOTS):
        issue_read(p, p)
    pltpu.sync_copy(zeros_hbm, acc.at[pl.ds(s * RPT, RPT)])
    plsc.subcore_barrier()

    def body(jj, carry):
        for p in range(SLOTS):
            t = jj * SLOTS + p
            _wait_write(enew_hbm, rbuf[p], rsem[p])
            pltpu.async_copy(rbuf[p], acc.at[didx_v.at[t]], asem[p],
                             add=True)
        for p in range(SLOTS):
            _wait_write(enew_hbm, rbuf[p], asem[p])
            issue_read(jj * SLOTS + p + SLOTS, p)
        return carry

    lax.fori_loop(0, BLKS - 1, body, 0)
    for p in range(SLOTS):
        t = (BLKS - 1) * SLOTS + p
        _wait_write(enew_hbm, rbuf[p], rsem[p])
        pltpu.async_copy(rbuf[p], acc.at[didx_v.at[t]], asem[p], add=True)
    for p in range(SLOTS):
        _wait_write(enew_hbm, rbuf[p], asem[p])
    plsc.subcore_barrier()
    pltpu.sync_copy(acc.at[pl.ds(s * RPT, RPT)],
                    out_hbm.at[c, pl.ds(s * RPT, RPT)])


@functools.lru_cache(maxsize=None)
def _sc_gather_call():
    return functools.partial(
        pl.kernel,
        mesh=plsc.VectorSubcoreMesh(core_axis_name="c", subcore_axis_name="s"),
        out_type=jax.ShapeDtypeStruct((EPAD, H), _F32),
        scratch_types=[pltpu.VMEM((KMAX, CH), jnp.int32),
                       pltpu.VMEM((KMAX, CH), jnp.int32)]
        + [pltpu.VMEM((CH, H), _F32)] * (2 * SLOTS)
        + [pltpu.SemaphoreType.DMA] * (2 * SLOTS),
    )(_sc_gather_body)


@functools.lru_cache(maxsize=None)
def _sc_scatter_call():
    return functools.partial(
        pl.kernel,
        mesh=plsc.VectorSubcoreMesh(core_axis_name="c", subcore_axis_name="s"),
        out_type=jax.ShapeDtypeStruct((2, NPAD, H), _F32),
        scratch_types=[pltpu.VMEM((NCH, CH), jnp.int32)]
        + [pltpu.VMEM((CH, H), _F32)] * SLOTS
        + [pltpu.VMEM_SHARED((NPAD, H), _F32)]
        + [pltpu.SemaphoreType.DMA] * (2 * SLOTS),
    )(_sc_scatter_body)


def kernel(graph_x, edge_index, edge_attr, velocity_sequence_noise,
           enc_node, enc_edge, mp_edge, mp_node, dec, norm_stats):
    del velocity_sequence_noise
    node_mean, node_std, edge_mean, edge_std, out_mean, out_std = norm_stats
    mp = len(mp_edge)

    src = edge_index[0].astype(jnp.int32)
    dst = edge_index[1].astype(jnp.int32)
    pad_e = EPAD - E
    sidx = jnp.concatenate(
        [src, jnp.full((pad_e,), TRASH, jnp.int32)]).reshape(EPAD // CH, CH)
    didx = jnp.concatenate(
        [dst, jnp.full((pad_e,), TRASH, jnp.int32)]).reshape(EPAD // CH, CH)
    ipad = jnp.full((KMAX, CH), TRASH, jnp.int32)
    sidx = jnp.concatenate([sidx, ipad])
    didx = jnp.concatenate([didx, ipad])

    gx8 = jnp.zeros((NPAD, 8), _F32).at[:N, :3].set(graph_x)
    ea8 = jnp.zeros((EPAD, 8), _F32).at[:E, :3].set(edge_attr)
    frames_pad = jnp.zeros((NPAD, H), _F32).at[:N, :2].set(graph_x[:, 1:3])
    zrows = jnp.zeros((RPT, H), _F32)

    def r(v):
        return v.reshape(1, H)

    nw1, nb1, nw2, nb2, nw3, nb3, nlg, nlb = enc_node
    ninv = 1.0 / node_std
    nw1p = jnp.zeros((16, H), _F32).at[:11].set(nw1 * ninv[:, None])
    nb1p = nb1 - (node_mean * ninv) @ nw1

    ew1, eb1, ew2, eb2, ew3, eb3, elg, elb = enc_edge
    einv = 1.0 / edge_std
    ew1p = jnp.zeros((8, H), _F32).at[:3].set(ew1 * einv[:, None])
    eb1p = eb1 - (edge_mean * einv) @ ew1

    d1, db1, d2, db2, d3, db3 = dec
    d3p = jnp.zeros((H, H), _F32).at[:, :2].set(d3)
    db3p = jnp.zeros((H,), _F32).at[:2].set(db3)
    scale = jnp.zeros((1, H), _F32).at[0, :2].set(out_std)
    shift = jnp.zeros((1, H), _F32).at[0, :2].set(out_mean)

    w1s0 = mp_edge[0][0][H:2 * H]
    w1d0 = mp_edge[0][0][2 * H:3 * H]
    hv, hs, hd = _node_enc_call(gx8, nw1p, r(nb1p), nw2, r(nb2), nw3, r(nb3),
                                r(nlg), r(nlb), w1s0, w1d0)
    he = _edge_enc_call(ea8, ew1p, r(eb1p), ew2, r(eb2), ew3, r(eb3),
                        r(elg), r(elb))

    pv = None
    for i in range(mp):
        we = mp_edge[i]
        wn = mp_node[i]
        g = _sc_gather_call()(hs, hd, sidx, didx)
        he = _edge_mlp_call(he, g, we[0][:H], r(we[1]), we[2], r(we[3]),
                            we[4], r(we[5]), r(we[6]), r(we[7]))
        parts = _sc_scatter_call()(he, didx, zrows)
        wv = wn[0][:H]
        wa = wn[0][H:]
        if i < mp - 1:
            w1s = mp_edge[i + 1][0][H:2 * H]
            w1d = mp_edge[i + 1][0][2 * H:3 * H]
            hv, hs, hd = _node_mp_call(hv, parts[0], parts[1], wv, wa,
                                       r(wn[1]), wn[2], r(wn[3]), wn[4],
                                       r(wn[5]), r(wn[6]), r(wn[7]),
                                       w1s, w1d)
        else:
            pv = _node_final_call(hv, parts[0], parts[1], frames_pad, wv, wa,
                                  r(wn[1]), wn[2], r(wn[3]), wn[4], r(wn[5]),
                                  r(wn[6]), r(wn[7]), d1, r(db1), d2, r(db2),
                                  d3p, r(db3p), scale, shift)

    return pv[:N, :2]


# speedup vs baseline: 1.2415x; 1.0007x over previous
import functools

import jax
import jax.numpy as jnp
from jax import lax
from jax.experimental import pallas as pl
from jax.experimental.pallas import tpu as pltpu
from jax.experimental.pallas import tpu_sc as plsc

N = 10000
E = 160000
H = 128
NPAD = 10240
EPAD = 163840
CH = 64
NCH = 80
SLOTS = 4
BLKS = NCH // SLOTS
KC0 = 112
KC1 = 48
KMAX = max(KC0, KC1)
RPT = NPAD // 16
TRASH = 10200
TE = 1024
TN = 512
_F32 = jnp.float32


def _ln(t, g, b):
    mu = jnp.mean(t, axis=-1, keepdims=True)
    var = jnp.mean((t - mu) ** 2, axis=-1, keepdims=True)
    return (t - mu) * lax.rsqrt(var + 1e-5) * g + b


def _dot(x, w):
    return jnp.dot(x.astype(jnp.bfloat16), w.astype(jnp.bfloat16),
                   preferred_element_type=_F32)


def _node_enc_body(gx, w1, b1, w2, b2, w3, b3, lg, lb, w1s, w1d,
                   hv_o, hs_o, hd_o):
    x = gx[...]
    t = x[:, 1:2] * w1[0:1, :] + x[:, 2:3] * w1[1:2, :] + b1[...]
    tp = x[:, 0:1]
    for k in range(9):
        t = t + jnp.where(tp == float(k), w1[2 + k:3 + k, :], 0.0)
    t = jax.nn.relu(t)
    t = jax.nn.relu(_dot(t, w2[...]) + b2[...])
    t = _dot(t, w3[...]) + b3[...]
    hv = _ln(t, lg[...], lb[...])
    hv_o[...] = hv
    hs_o[...] = _dot(hv, w1s[...])
    hd_o[...] = _dot(hv, w1d[...])


def _edge_enc_body(ea, w1, b1, w2, b2, w3, b3, lg, lb, he_o):
    x = ea[...]
    t = _dot(x, w1[...]) + b1[...]
    t = jax.nn.relu(t)
    t = jax.nn.relu(_dot(t, w2[...]) + b2[...])
    t = _dot(t, w3[...]) + b3[...]
    he_o[...] = _ln(t, lg[...], lb[...])


def _edge_mlp_body(he, g, w1e, b1, w2, b2, w3, b3, lg, lb, out):
    x = he[...]
    t = jax.nn.relu(_dot(x, w1e[...]) + g[...] + b1[...])
    t = jax.nn.relu(_dot(t, w2[...]) + b2[...])
    t = _dot(t, w3[...]) + b3[...]
    out[...] = _ln(t, lg[...], lb[...]) + x


def _node_mp_body(hv, p0, p1, wv, wa, b1, w2, b2, w3, b3, lg, lb, w1s, w1d,
                  hv_o, hs_o, hd_o):
    x = hv[...]
    agg = p0[...] + p1[...]
    t = jax.nn.relu(_dot(x, wv[...]) + _dot(agg, wa[...]) + b1[...])
    t = jax.nn.relu(_dot(t, w2[...]) + b2[...])
    t = _dot(t, w3[...]) + b3[...]
    v = _ln(t, lg[...], lb[...]) + x
    hv_o[...] = v
    hs_o[...] = _dot(v, w1s[...])
    hd_o[...] = _dot(v, w1d[...])


def _node_final_body(hv, p0, p1, fr, wv, wa, b1, w2, b2, w3, b3, lg, lb,
                     d1, db1, d2, db2, d3, db3, scale, shift, pv_o):
    x = hv[...]
    agg = p0[...] + p1[...]
    t = jax.nn.relu(_dot(x, wv[...]) + _dot(agg, wa[...]) + b1[...])
    t = jax.nn.relu(_dot(t, w2[...]) + b2[...])
    t = _dot(t, w3[...]) + b3[...]
    v = _ln(t, lg[...], lb[...]) + x
    t = jax.nn.relu(_dot(v, d1[...]) + db1[...])
    t = jax.nn.relu(_dot(t, d2[...]) + db2[...])
    o = _dot(t, d3[...]) + db3[...]
    pv_o[...] = fr[...] + o * scale[...] + shift[...]


def _row_spec(t):
    return pl.BlockSpec((t, H), lambda i: (i, 0))


def _const_spec(shape):
    nd = len(shape)
    return pl.BlockSpec(shape, lambda i: (0,) * nd)


def _small_spec(t, w):
    return pl.BlockSpec((t, w), lambda i: (i, 0))


def _node_enc_call(gx8, w1, b1, w2, b2, w3, b3, lg, lb, w1s, w1d):
    grid = (NPAD // TN,)
    hh = _const_spec((H, H))
    v = _const_spec((1, H))
    return pl.pallas_call(
        _node_enc_body,
        grid=grid,
        in_specs=[_small_spec(TN, 8), _const_spec((16, H)), v, hh, v, hh, v,
                  v, v, hh, hh],
        out_specs=[_row_spec(TN)] * 3,
        out_shape=[jax.ShapeDtypeStruct((NPAD, H), _F32)] * 3,
    )(gx8, w1, b1, w2, b2, w3, b3, lg, lb, w1s, w1d)


def _edge_enc_call(ea8, w1, b1, w2, b2, w3, b3, lg, lb):
    grid = (EPAD // TE,)
    hh = _const_spec((H, H))
    v = _const_spec((1, H))
    return pl.pallas_call(
        _edge_enc_body,
        grid=grid,
        in_specs=[_small_spec(TE, 8), _const_spec((8, H)), v, hh, v, hh, v,
                  v, v],
        out_specs=_row_spec(TE),
        out_shape=jax.ShapeDtypeStruct((EPAD, H), _F32),
    )(ea8, w1, b1, w2, b2, w3, b3, lg, lb)


def _edge_mlp_call(he, g, w1e, b1, w2, b2, w3, b3, lg, lb):
    grid = (EPAD // TE,)
    hh = _const_spec((H, H))
    v = _const_spec((1, H))
    return pl.pallas_call(
        _edge_mlp_body,
        grid=grid,
        in_specs=[_row_spec(TE)] * 2 + [hh, v, hh, v, hh, v, v, v],
        out_specs=_row_spec(TE),
        out_shape=jax.ShapeDtypeStruct((EPAD, H), _F32),
    )(he, g, w1e, b1, w2, b2, w3, b3, lg, lb)


def _node_mp_call(hv, p0, p1, wv, wa, b1, w2, b2, w3, b3, lg, lb, w1s, w1d):
    grid = (NPAD // TN,)
    hh = _const_spec((H, H))
    v = _const_spec((1, H))
    return pl.pallas_call(
        _node_mp_body,
        grid=grid,
        in_specs=[_row_spec(TN)] * 3 + [hh, hh, v, hh, v, hh, v, v, v, hh, hh],
        out_specs=[_row_spec(TN)] * 3,
        out_shape=[jax.ShapeDtypeStruct((NPAD, H), _F32)] * 3,
    )(hv, p0, p1, wv, wa, b1, w2, b2, w3, b3, lg, lb, w1s, w1d)


def _node_final_call(hv, p0, p1, fr, wv, wa, b1, w2, b2, w3, b3, lg, lb,
                     d1, db1, d2, db2, d3, db3, scale, shift):
    grid = (NPAD // TN,)
    hh = _const_spec((H, H))
    v = _const_spec((1, H))
    return pl.pallas_call(
        _node_final_body,
        grid=grid,
        in_specs=[_row_spec(TN)] * 4
        + [hh, hh, v, hh, v, hh, v, v, v, hh, v, hh, v, hh, v, v, v],
        out_specs=_row_spec(TN),
        out_shape=jax.ShapeDtypeStruct((NPAD, H), _F32),
    )(hv, p0, p1, fr, wv, wa, b1, w2, b2, w3, b3, lg, lb,
      d1, db1, d2, db2, d3, db3, scale, shift)


def _wait_write(hbm, buf, sem):
    pltpu.make_async_copy(hbm.at[pl.ds(0, CH)], buf, sem).wait()


def _add_rows(a, b):
    def body(r2, carry):
        for u in range(2):
            r = r2 * 2 + u
            for q in range(H // 16):
                sl = pl.ds(q * 16, 16)
                a[r, sl] = a[r, sl] + b[r, sl]
        return carry

    lax.fori_loop(0, CH // 2, body, 0)


def _sc_gather_body(hs_hbm, hd_hbm, sidx_hbm, didx_hbm, g_hbm,
                    sidx_v, didx_v,
                    a0, a1, a2, a3, b0, b1, b2, b3,
                    g0, g1, g2, g3, w0, w1, w2, w3):
    c = lax.axis_index("c")
    s = lax.axis_index("s")
    cbase = jnp.where(c == 0, s * KC0, 16 * KC0 + s * KC1)
    nblk = jnp.where(c == 0, KC0 // SLOTS, KC1 // SLOTS)
    pltpu.sync_copy(sidx_hbm.at[pl.ds(cbase, KMAX)], sidx_v)
    pltpu.sync_copy(didx_hbm.at[pl.ds(cbase, KMAX)], didx_v)
    abuf = (a0, a1, a2, a3)
    bbuf = (b0, b1, b2, b3)
    gsem = (g0, g1, g2, g3)
    wsem = (w0, w1, w2, w3)

    def issue(t, p):
        pltpu.async_copy(hs_hbm.at[sidx_v.at[t]], abuf[p], gsem[p])
        pltpu.async_copy(hd_hbm.at[didx_v.at[t]], bbuf[p], gsem[p])

    def wait_g(p):
        _wait_write(hs_hbm, abuf[p], gsem[p])
        _wait_write(hs_hbm, bbuf[p], gsem[p])

    def flush(t, p):
        wait_g(p)
        _add_rows(abuf[p], bbuf[p])
        pltpu.async_copy(abuf[p], g_hbm.at[pl.ds((cbase + t) * CH, CH)],
                         wsem[p])

    for p in range(SLOTS):
        issue(p, p)

    def body(jj, carry):
        for p in range(SLOTS):
            flush(jj * SLOTS + p, p)
        for p in range(SLOTS):
            _wait_write(g_hbm, abuf[p], wsem[p])
            issue(jj * SLOTS + p + SLOTS, p)
        return carry

    lax.fori_loop(0, nblk - 1, body, 0)
    for p in range(SLOTS):
        flush((nblk - 1) * SLOTS + p, p)
    for p in range(SLOTS):
        _wait_write(g_hbm, abuf[p], wsem[p])


def _sc_scatter_body(enew_hbm, didx_hbm, zeros_hbm, out_hbm,
                     didx_v, r0, r1, r2, r3, acc,
                     rs0, rs1, rs2, rs3, as0, as1, as2, as3):
    c = lax.axis_index("c")
    s = lax.axis_index("s")
    wid = s * 2 + c
    pltpu.sync_copy(didx_hbm.at[pl.ds(wid * NCH, NCH)], didx_v)
    base = wid * NCH * CH
    rbuf = (r0, r1, r2, r3)
    rsem = (rs0, rs1, rs2, rs3)
    asem = (as0, as1, as2, as3)

    def issue_read(t, p):
        pltpu.async_copy(enew_hbm.at[pl.ds(base + t * CH, CH)], rbuf[p],
                         rsem[p])

    for p in range(SLOTS):
        issue_read(p, p)
    pltpu.sync_copy(zeros_hbm, acc.at[pl.ds(s * RPT, RPT)])
    plsc.subcore_barrier()

    def body(jj, carry):
        for p in range(SLOTS):
            t = jj * SLOTS + p
            _wait_write(enew_hbm, rbuf[p], rsem[p])
            pltpu.async_copy(rbuf[p], acc.at[didx_v.at[t]], asem[p],
                             add=True)
        for p in range(SLOTS):
            _wait_write(enew_hbm, rbuf[p], asem[p])
            issue_read(jj * SLOTS + p + SLOTS, p)
        return carry

    lax.fori_loop(0, BLKS - 1, body, 0)
    for p in range(SLOTS):
        t = (BLKS - 1) * SLOTS + p
        _wait_write(enew_hbm, rbuf[p], rsem[p])
        pltpu.async_copy(rbuf[p], acc.at[didx_v.at[t]], asem[p], add=True)
    for p in range(SLOTS):
        _wait_write(enew_hbm, rbuf[p], asem[p])
    plsc.subcore_barrier()
    pltpu.sync_copy(acc.at[pl.ds(s * RPT, RPT)],
                    out_hbm.at[c, pl.ds(s * RPT, RPT)])


@functools.lru_cache(maxsize=None)
def _sc_gather_call():
    return functools.partial(
        pl.kernel,
        mesh=plsc.VectorSubcoreMesh(core_axis_name="c", subcore_axis_name="s"),
        out_type=jax.ShapeDtypeStruct((EPAD, H), _F32),
        scratch_types=[pltpu.VMEM((KMAX, CH), jnp.int32),
                       pltpu.VMEM((KMAX, CH), jnp.int32)]
        + [pltpu.VMEM((CH, H), _F32)] * (2 * SLOTS)
        + [pltpu.SemaphoreType.DMA] * (2 * SLOTS),
    )(_sc_gather_body)


@functools.lru_cache(maxsize=None)
def _sc_scatter_call():
    return functools.partial(
        pl.kernel,
        mesh=plsc.VectorSubcoreMesh(core_axis_name="c", subcore_axis_name="s"),
        out_type=jax.ShapeDtypeStruct((2, NPAD, H), _F32),
        scratch_types=[pltpu.VMEM((NCH, CH), jnp.int32)]
        + [pltpu.VMEM((CH, H), _F32)] * SLOTS
        + [pltpu.VMEM_SHARED((NPAD, H), _F32)]
        + [pltpu.SemaphoreType.DMA] * (2 * SLOTS),
    )(_sc_scatter_body)


def kernel(graph_x, edge_index, edge_attr, velocity_sequence_noise,
           enc_node, enc_edge, mp_edge, mp_node, dec, norm_stats):
    del velocity_sequence_noise
    node_mean, node_std, edge_mean, edge_std, out_mean, out_std = norm_stats
    mp = len(mp_edge)

    src = edge_index[0].astype(jnp.int32)
    dst = edge_index[1].astype(jnp.int32)
    pad_e = EPAD - E
    sidx = jnp.concatenate(
        [src, jnp.full((pad_e,), TRASH, jnp.int32)]).reshape(EPAD // CH, CH)
    didx = jnp.concatenate(
        [dst, jnp.full((pad_e,), TRASH, jnp.int32)]).reshape(EPAD // CH, CH)
    ipad = jnp.full((KMAX, CH), TRASH, jnp.int32)
    sidx = jnp.concatenate([sidx, ipad])
    didx = jnp.concatenate([didx, ipad])

    gx8 = jnp.zeros((NPAD, 8), _F32).at[:N, :3].set(graph_x)
    ea8 = jnp.zeros((EPAD, 8), _F32).at[:E, :3].set(edge_attr)
    frames_pad = jnp.zeros((NPAD, H), _F32).at[:N, :2].set(graph_x[:, 1:3])
    zrows = jnp.zeros((RPT, H), _F32)

    def r(v):
        return v.reshape(1, H)

    nw1, nb1, nw2, nb2, nw3, nb3, nlg, nlb = enc_node
    ninv = 1.0 / node_std
    nw1p = jnp.zeros((16, H), _F32).at[:11].set(nw1 * ninv[:, None])
    nb1p = nb1 - (node_mean * ninv) @ nw1

    ew1, eb1, ew2, eb2, ew3, eb3, elg, elb = enc_edge
    einv = 1.0 / edge_std
    ew1p = jnp.zeros((8, H), _F32).at[:3].set(ew1 * einv[:, None])
    eb1p = eb1 - (edge_mean * einv) @ ew1

    d1, db1, d2, db2, d3, db3 = dec
    d3p = jnp.zeros((H, H), _F32).at[:, :2].set(d3)
    db3p = jnp.zeros((H,), _F32).at[:2].set(db3)
    scale = jnp.zeros((1, H), _F32).at[0, :2].set(out_std)
    shift = jnp.zeros((1, H), _F32).at[0, :2].set(out_mean)

    w1s0 = mp_edge[0][0][H:2 * H]
    w1d0 = mp_edge[0][0][2 * H:3 * H]
    hv, hs, hd = _node_enc_call(gx8, nw1p, r(nb1p), nw2, r(nb2), nw3, r(nb3),
                                r(nlg), r(nlb), w1s0, w1d0)
    he = _edge_enc_call(ea8, ew1p, r(eb1p), ew2, r(eb2), ew3, r(eb3),
                        r(elg), r(elb))

    pv = None
    for i in range(mp):
        we = mp_edge[i]
        wn = mp_node[i]
        g = _sc_gather_call()(hs, hd, sidx, didx)
        he = _edge_mlp_call(he, g, we[0][:H], r(we[1]), we[2], r(we[3]),
                            we[4], r(we[5]), r(we[6]), r(we[7]))
        parts = _sc_scatter_call()(he, didx, zrows)
        wv = wn[0][:H]
        wa = wn[0][H:]
        if i < mp - 1:
            w1s = mp_edge[i + 1][0][H:2 * H]
            w1d = mp_edge[i + 1][0][2 * H:3 * H]
            hv, hs, hd = _node_mp_call(hv, parts[0], parts[1], wv, wa,
                                       r(wn[1]), wn[2], r(wn[3]), wn[4],
                                       r(wn[5]), r(wn[6]), r(wn[7]),
                                       w1s, w1d)
        else:
            pv = _node_final_call(hv, parts[0], parts[1], frames_pad, wv, wa,
                                  r(wn[1]), wn[2], r(wn[3]), wn[4], r(wn[5]),
                                  r(wn[6]), r(wn[7]), d1, r(db1), d2, r(db2),
                                  d3p, r(db3p), scale, shift)

    return pv[:N, :2]
